# Initial kernel scaffold; baseline (speedup 1.0000x reference)
import jax
import jax.numpy as jnp
from jax.experimental import pallas as pl


def kernel(users, occupations, items, user_embedding, item_embedding, occupation_embedding, user_bias, item_bias, bias):
    raise NotImplementedError("write your pallas kernel here")


# speedup vs baseline: 2.4094x; 2.4094x over previous
import functools

import jax
import jax.numpy as jnp
from jax import lax
from jax.experimental import pallas as pl
from jax.experimental.pallas import tpu as pltpu
from jax.experimental.pallas import tpu_sc as plsc


def _build(B, L, D, NC, NS):
    NW = NC * NS
    UPW = B // NW
    LP = -(-L // 8) * 8
    NSL = D // 16
    offs = [16 * k for k in range(L // 16)]
    if L % 16:
        offs.append(L - 16)
    pad_start = LP - 16

    mesh = plsc.VectorSubcoreMesh(core_axis_name="c", subcore_axis_name="s")

    @functools.partial(
        pl.kernel,
        out_type=jax.ShapeDtypeStruct((B * L,), jnp.float32),
        mesh=mesh,
        compiler_params=pltpu.CompilerParams(needs_layout_passes=False),
        scratch_types=[
            pltpu.VMEM((UPW,), jnp.int32),
            pltpu.VMEM((UPW,), jnp.int32),
            pltpu.VMEM((UPW, D), jnp.float32),
            pltpu.VMEM((UPW, D), jnp.float32),
            pltpu.VMEM((UPW + 16,), jnp.float32),
            pltpu.VMEM((16,), jnp.float32),
            pltpu.VMEM((UPW * L,), jnp.int32),
            pltpu.VMEM((UPW, LP), jnp.int32),
            pltpu.VMEM((LP,), jnp.float32),
            pltpu.VMEM((LP, D), jnp.float32),
            pltpu.VMEM((16, 16), jnp.float32),
            pltpu.VMEM((UPW * L,), jnp.float32),
            pltpu.SemaphoreType.DMA,
            pltpu.SemaphoreType.DMA,
        ],
    )
    def k(users_r, occ_r, items_r, ue_r, ie_r, oe_r, ub_r, ib_r, bias_r,
          out_r,
          uidx_v, oidx_v, q_v, oe_v, ub_v, bias_v, items_f_v, items_p, ib_v,
          rows_v, tbuf, out_v, sem_a, sem_b):
        wid = lax.axis_index("s") * NC + lax.axis_index("c")
        base = wid * UPW
        iota = lax.iota(jnp.int32, 16)

        pltpu.sync_copy(users_r.at[pl.ds(base, UPW)], uidx_v)
        pltpu.sync_copy(occ_r.at[pl.ds(base, UPW)], oidx_v)
        pltpu.sync_copy(items_r.at[pl.ds(base * L, UPW * L)], items_f_v)
        pltpu.sync_copy(bias_r, bias_v.at[pl.ds(0, 1)])
        pltpu.async_copy(ub_r.at[uidx_v], ub_v.at[pl.ds(0, UPW)], sem_a).wait()
        pltpu.async_copy(ue_r.at[uidx_v], q_v, sem_a).wait()
        pltpu.async_copy(oe_r.at[oidx_v], oe_v, sem_a).wait()

        def add_oe(i, _):
            b = i // NSL
            s = (i % NSL) * 16
            q_v[b, pl.ds(s, 16)] = q_v[b, pl.ds(s, 16)] + oe_v[b, pl.ds(s, 16)]
            return 0
        lax.fori_loop(0, UPW * NSL, add_oe, 0)

        def fill(b, _):
            for off in offs:
                items_p[b, pl.ds(off, 16)] = items_f_v[pl.ds(b * L + off, 16)]
            if LP != L:
                v = items_p[b, pl.ds(pad_start, 16)]
                items_p[b, pl.ds(pad_start, 16)] = jnp.where(
                    iota >= (L - pad_start), 0, v)
            return 0
        lax.fori_loop(0, UPW, fill, 0)

        bias0 = bias_v[...][0]

        def user_body(b, _):
            idx = items_p.at[b]
            h1 = pltpu.async_copy(ie_r.at[idx], rows_v, sem_a)
            h2 = pltpu.async_copy(ib_r.at[idx], ib_v, sem_b)
            h1.wait()
            h2.wait()
            qs = [q_v[b, pl.ds(16 * s, 16)] for s in range(NSL)]
            ubb = ub_v[pl.ds(b, 16)][0] + bias0
            for off in offs:
                for i in range(16):
                    acc = rows_v[off + i, pl.ds(0, 16)] * qs[0]
                    for s in range(1, NSL):
                        acc = acc + rows_v[off + i, pl.ds(16 * s, 16)] * qs[s]
                    tbuf[i, :] = acc
                svec = plsc.load_gather(tbuf, [iota, jnp.zeros((16,), jnp.int32)])
                for j in range(1, 16):
                    svec = svec + plsc.load_gather(
                        tbuf, [iota, jnp.full((16,), j, jnp.int32)])
                ib16 = plsc.load_gather(ib_v, [off + iota])
                res = svec + ib16 + ubb
                out_v[pl.ds(b * L + off, 16)] = res
            return 0
        lax.fori_loop(0, UPW, user_body, 0)

        pltpu.sync_copy(out_v, out_r.at[pl.ds(base * L, UPW * L)])

    return k


def kernel(users, occupations, items, user_embedding, item_embedding,
           occupation_embedding, user_bias, item_bias, bias):
    B, L = items.shape
    D = user_embedding.shape[1]
    info = plsc.get_sparse_core_info()
    k = _build(B, L, D, info.num_cores, info.num_subcores)
    out = k(users, occupations, items.reshape(-1), user_embedding,
            item_embedding, occupation_embedding, user_bias, item_bias, bias)
    return out.reshape(B, L)
